# speedup vs baseline: 4.8620x; 4.8620x over previous
import functools

import jax
import jax.numpy as jnp
from jax import lax
from jax.experimental import pallas as pl
from jax.experimental.pallas import tpu as pltpu
from jax.experimental.pallas import tpu_sc as plsc

N_NODES = 10000
D = 128

NC = 2
NS = 16
NW = NC * NS
CHUNK = 128

N_PAD = 10240
ROWS_PER_TILE = N_PAD // NS


def _sc_aggregate_body(with_counts, *refs):
    if with_counts:
        (table, src_hbm, dst_hbm, part_hbm, cnt_hbm,
         src_v, dst_v, rows, zbuf, ones_v, zcnt, acc, cacc, sem) = refs
    else:
        (table, src_hbm, dst_hbm, part_hbm,
         src_v, dst_v, rows, zbuf, acc, sem) = refs

    c = lax.axis_index("c")
    s = lax.axis_index("s")
    wid = c * NS + s
    ec = src_v.shape[0]

    for i in range(16):
        for j in range(D // 16):
            zbuf[i, pl.ds(j * 16, 16)] = jnp.zeros((16,), jnp.float32)
    row0 = s * ROWS_PER_TILE
    for k in range(ROWS_PER_TILE // 16):
        pltpu.sync_copy(zbuf, acc.at[pl.ds(row0 + k * 16, 16)])
    if with_counts:
        for k in range(ROWS_PER_TILE // 16):
            zcnt[pl.ds(k * 16, 16)] = jnp.zeros((16,), jnp.float32)
        pltpu.sync_copy(zcnt, cacc.at[pl.ds(row0, ROWS_PER_TILE)])
        for k in range(CHUNK // 16):
            ones_v[pl.ds(k * 16, 16)] = jnp.ones((16,), jnp.float32)
    plsc.subcore_barrier()

    pltpu.sync_copy(src_hbm.at[wid], src_v)
    pltpu.sync_copy(dst_hbm.at[wid], dst_v)

    def body(j, carry):
        pltpu.async_copy(table.at[src_v.at[j]], rows, sem).wait()
        pltpu.sync_copy(rows, acc.at[dst_v.at[j]], add=True)
        if with_counts:
            pltpu.sync_copy(ones_v, cacc.at[dst_v.at[j]], add=True)
        return carry

    lax.fori_loop(0, ec, body, 0)
    plsc.subcore_barrier()

    pltpu.sync_copy(acc.at[pl.ds(row0, ROWS_PER_TILE)],
                    part_hbm.at[c, pl.ds(row0, ROWS_PER_TILE)])
    if with_counts:
        pltpu.sync_copy(cacc.at[pl.ds(row0, ROWS_PER_TILE)],
                        cnt_hbm.at[c, pl.ds(row0, ROWS_PER_TILE)])


def _make_sc_aggregate(ec, with_counts):
    mesh = plsc.VectorSubcoreMesh(core_axis_name="c", subcore_axis_name="s",
                                  num_cores=NC, num_subcores=NS)
    out_type = [jax.ShapeDtypeStruct((NC, N_PAD, D), jnp.float32)]
    if with_counts:
        out_type.append(jax.ShapeDtypeStruct((NC, N_PAD), jnp.float32))
    scratch = [
        pltpu.VMEM((ec, CHUNK), jnp.int32),
        pltpu.VMEM((ec, CHUNK), jnp.int32),
        pltpu.VMEM((CHUNK, D), jnp.float32),
        pltpu.VMEM((16, D), jnp.float32),
    ]
    if with_counts:
        scratch += [
            pltpu.VMEM((CHUNK,), jnp.float32),
            pltpu.VMEM((ROWS_PER_TILE,), jnp.float32),
        ]
    scratch += [pltpu.VMEM_SHARED((N_PAD, D), jnp.float32)]
    if with_counts:
        scratch += [pltpu.VMEM_SHARED((N_PAD,), jnp.float32)]
    scratch += [pltpu.SemaphoreType.DMA]

    return pl.kernel(
        functools.partial(_sc_aggregate_body, with_counts),
        out_type=out_type, mesh=mesh, scratch_types=scratch,
        name="sage_sc_agg" + ("_cnt" if with_counts else ""))


def _tc_linear_body(relu, p0, p1, c0, c1, x, wl, wr, b, out):
    inv = 1.0 / jnp.maximum(c0[...] + c1[...], 1.0)
    agg = (p0[...] + p1[...]) * inv
    y = (jnp.dot(agg, wl[...], preferred_element_type=jnp.float32)
         + jnp.dot(x[...], wr[...], preferred_element_type=jnp.float32)
         + b[...])
    if relu:
        y = jnp.maximum(y, 0.0)
    out[...] = y


def _make_tc_linear(relu, rows_blk=1024):
    grid = (N_PAD // rows_blk,)
    row_spec = pl.BlockSpec((rows_blk, D), lambda i: (i, 0))
    cnt_spec = pl.BlockSpec((rows_blk, 1), lambda i: (i, 0))
    full = pl.BlockSpec((D, D), lambda i: (0, 0))
    bias = pl.BlockSpec((1, D), lambda i: (0, 0))
    return pl.pallas_call(
        functools.partial(_tc_linear_body, relu),
        grid=grid,
        in_specs=[row_spec, row_spec, cnt_spec, cnt_spec, row_spec, full,
                  full, bias],
        out_specs=row_spec,
        out_shape=jax.ShapeDtypeStruct((N_PAD, D), jnp.float32),
        name="sage_tc_linear" + ("_relu" if relu else ""))


@jax.jit
def kernel(x, edge_index, W_l1, W_r1, b1, W_l2, W_r2, b2):
    n_edges = edge_index.shape[1]
    per_tile = -(-n_edges // (NW * CHUNK)) * CHUNK
    e_pad = per_tile * NW
    ec = per_tile // CHUNK

    src = edge_index[0].astype(jnp.int32)
    dst = edge_index[1].astype(jnp.int32)
    pad = e_pad - n_edges
    src = jnp.concatenate([src, jnp.zeros((pad,), jnp.int32)])
    dst = jnp.concatenate([dst, jnp.full((pad,), N_NODES, jnp.int32)])
    src = src.reshape(NW, ec, CHUNK)
    dst = dst.reshape(NW, ec, CHUNK)

    x_pad = jnp.zeros((N_PAD, D), jnp.float32).at[:N_NODES].set(x)

    agg1 = _make_sc_aggregate(ec, True)
    agg2 = _make_sc_aggregate(ec, False)
    lin1 = _make_tc_linear(True)
    lin2 = _make_tc_linear(False)

    part1, cnt = agg1(x_pad, src, dst)
    c0 = cnt[0].reshape(N_PAD, 1)
    c1 = cnt[1].reshape(N_PAD, 1)
    h = lin1(part1[0], part1[1], c0, c1, x_pad, W_l1, W_r1,
             b1.reshape(1, D))
    (part2,) = agg2(h, src, dst)
    out = lin2(part2[0], part2[1], c0, c1, h, W_l2, W_r2,
               b2.reshape(1, D))
    return out[:N_NODES]
